# Initial kernel scaffold; baseline (speedup 1.0000x reference)
import jax
import jax.numpy as jnp
from jax.experimental import pallas as pl


def kernel(inputs, indices, output_shape):
    raise NotImplementedError("write your pallas kernel here")


# speedup vs baseline: 15.1349x; 15.1349x over previous
import functools

import jax
import jax.numpy as jnp
from jax import lax
from jax.experimental import pallas as pl
from jax.experimental.pallas import tpu as pltpu
from jax.experimental.pallas import tpu_sc as plsc

B, PH, PW, C = 4, 112, 112, 96
OH, OW = 224, 224

EPB = PH * PW * C
OPB = OH * OW * C
NRANGE = 4
RANGE = OPB // NRANGE
PAD = 1024
TPW = EPB // 16
CHUNK = 4704
NCHUNK = TPW // CHUNK
VPC = CHUNK // 16
TOTAL_IN = B * EPB
TOTAL_OUT = B * OPB

def _unpool_body(
    idx_hbm, val_hbm, out_hbm, acc, idxb0, idxb1, valb0, valb1, zbuf, sem_i, sem_v
):
    idxb = [idxb0, idxb1]
    valb = [valb0, valb1]
    c = lax.axis_index("c")
    s = lax.axis_index("s")
    iota = lax.iota(jnp.int32, 16)

    def _zb(i, _):
        zbuf[pl.ds(i * 16, 16)] = jnp.zeros((16,), jnp.float32)
        return _

    lax.fori_loop(0, CHUNK // 16, _zb, None)

    for p in range(B * 2):
        b = p // 2
        j = p % 2
        rid = 2 * c + j
        rbase = rid * RANGE
        in_base = b * EPB + s * TPW
        out_base = b * OPB + rbase + s * (RANGE // 16)

        def _zero(k, _):
            pltpu.sync_copy(
                zbuf, acc.at[pl.ds(s * (RANGE // 16) + k * CHUNK, CHUNK)]
            )
            return _

        lax.fori_loop(0, NCHUNK, _zero, None)
        plsc.subcore_barrier()

        def _load(k, bi):
            src = pl.ds(in_base + k * CHUNK, CHUNK)
            pltpu.make_async_copy(idx_hbm.at[src], idxb[bi], sem_i).start()
            pltpu.make_async_copy(val_hbm.at[src], valb[bi], sem_v).start()

        _load(0, 0)

        def _chunk(kk, _):
            for bi in range(2):
                k = 2 * kk + bi
                pltpu.make_async_copy(
                    idx_hbm.at[pl.ds(0, CHUNK)], idxb[bi], sem_i
                ).wait()
                pltpu.make_async_copy(
                    val_hbm.at[pl.ds(0, CHUNK)], valb[bi], sem_v
                ).wait()

                @pl.when(k + 1 < NCHUNK)
                def _():
                    _load(k + 1, 1 - bi)

                def _vec(i, _):
                    sl = pl.ds(i * 16, 16)
                    local = idxb[bi][sl] - rbase
                    m = lax.bitcast_convert_type(local, jnp.uint32) < jnp.uint32(RANGE)
                    dump = iota + (RANGE + (i * 16) % PAD)
                    idxb[bi][sl] = jnp.where(m, local, dump)
                    valb[bi][sl] = jnp.where(m, valb[bi][sl], 0.0)
                    return _

                lax.fori_loop(0, VPC, _vec, None)
                pltpu.sync_copy(valb[bi], acc.at[idxb[bi]], add=True)
            return _

        lax.fori_loop(0, NCHUNK // 2, _chunk, None)
        plsc.subcore_barrier()

        pltpu.sync_copy(
            acc.at[pl.ds(s * (RANGE // 16), RANGE // 16)],
            out_hbm.at[pl.ds(out_base, RANGE // 16)],
        )
        plsc.subcore_barrier()


@functools.cache
def _unpool():
    mesh = plsc.VectorSubcoreMesh(core_axis_name="c", subcore_axis_name="s")
    return pl.kernel(
        _unpool_body,
        out_type=jax.ShapeDtypeStruct((TOTAL_OUT,), jnp.float32),
        mesh=mesh,
        scratch_types=[
            pltpu.VMEM_SHARED((RANGE + PAD,), jnp.float32),
            pltpu.VMEM((CHUNK,), jnp.int32),
            pltpu.VMEM((CHUNK,), jnp.int32),
            pltpu.VMEM((CHUNK,), jnp.float32),
            pltpu.VMEM((CHUNK,), jnp.float32),
            pltpu.VMEM((CHUNK,), jnp.float32),
            pltpu.SemaphoreType.DMA,
            pltpu.SemaphoreType.DMA,
        ],
    )


def kernel(inputs, indices, output_shape):
    idx_flat = indices.reshape(-1).astype(jnp.int32)
    val_flat = inputs.reshape(-1)
    out = _unpool()(idx_flat, val_flat)
    return out.reshape(B, OH, OW, C)
